# baseline (speedup 1.0000x reference)
import functools
import math

import jax
import jax.numpy as jnp
from jax import lax
from jax.experimental import pallas as pl
from jax.experimental.pallas import tpu as pltpu
from jax.experimental.pallas import tpu_sc as plsc

_CHUNK = 128


def _sc_gather(table, idx3):
    nw, nch, chunk = idx3.shape
    d = table.shape[1]
    info = plsc.get_sparse_core_info()
    ncores = info.num_cores

    mesh = plsc.VectorSubcoreMesh(core_axis_name="c", subcore_axis_name="s")

    @functools.partial(
        pl.kernel,
        mesh=mesh,
        compiler_params=pltpu.CompilerParams(use_tc_tiling_on_sc=False),
        out_type=jax.ShapeDtypeStruct((nw, nch, chunk, d), jnp.float32),
        scratch_types=[
            pltpu.VMEM((nch, chunk), jnp.int32),
            pltpu.VMEM((chunk, d), jnp.float32),
            pltpu.VMEM((chunk, d), jnp.float32),
            pltpu.SemaphoreType.DMA,
            pltpu.SemaphoreType.DMA,
            pltpu.SemaphoreType.DMA,
            pltpu.SemaphoreType.DMA,
            pltpu.SemaphoreType.DMA,
        ],
    )
    def k(table_hbm, idx_hbm, out_hbm, idx_v, rows0, rows1,
          gsem0, gsem1, osem0, osem1, isem):
        wid = lax.axis_index("s") * ncores + lax.axis_index("c")

        pltpu.make_async_copy(idx_hbm.at[wid], idx_v, isem).start()
        pltpu.make_async_copy(idx_hbm.at[wid], idx_v, isem).wait()

        def start_gather(c, buf, sem):
            pltpu.make_async_copy(table_hbm.at[idx_v.at[c]], buf, sem).start()

        def wait_gather(c, buf, sem):
            pltpu.make_async_copy(table_hbm.at[idx_v.at[c]], buf, sem).wait()

        def start_out(c, buf, sem):
            pltpu.make_async_copy(buf, out_hbm.at[wid, c], sem).start()

        def wait_out(c, buf, sem):
            pltpu.make_async_copy(buf, out_hbm.at[wid, c], sem).wait()

        start_gather(0, rows0, gsem0)

        def step(c, buf, gsem, osem, nbuf, ngsem, nosem):
            @pl.when(c + 1 < nch)
            def _():
                @pl.when(c >= 1)
                def _():
                    wait_out(c - 1, nbuf, nosem)
                start_gather(c + 1, nbuf, ngsem)

            wait_gather(c, buf, gsem)
            start_out(c, buf, osem)

        def pair(p, carry):
            step(2 * p, rows0, gsem0, osem0, rows1, gsem1, osem1)
            step(2 * p + 1, rows1, gsem1, osem1, rows0, gsem0, osem0)
            return carry

        lax.fori_loop(0, nch // 2, pair, 0)

        wait_out(nch - 2, rows0, osem0)
        wait_out(nch - 1, rows1, osem1)

    return k(table, idx3)


def _tc_finish(g2, pe_rep, w2, c2, block_rows):
    m = g2.shape[0]
    grid = m // block_rows
    scale = 8.0

    def body(g_ref, pe_ref, w_ref, c_ref, o_ref):
        x = g_ref[...] * scale + pe_ref[...]
        r = jnp.dot(x, w_ref[...], preferred_element_type=jnp.float32)
        o_ref[...] = x * (r + c_ref[...])

    return pl.pallas_call(
        body,
        grid=(grid,),
        in_specs=[
            pl.BlockSpec((block_rows, 128), lambda i: (i, 0)),
            pl.BlockSpec((block_rows, 128), lambda i: (0, 0)),
            pl.BlockSpec((128, 128), lambda i: (0, 0)),
            pl.BlockSpec((1, 128), lambda i: (0, 0)),
        ],
        out_specs=pl.BlockSpec((block_rows, 128), lambda i: (i, 0)),
        out_shape=jax.ShapeDtypeStruct((m, 128), jnp.float32),
    )(g2, pe_rep, w2, c2)


def kernel(src, tok_embedding, W, b, pe):
    bsz, seqlen = src.shape
    d = tok_embedding.shape[1]
    n = bsz * seqlen

    nw = 32
    nch = n // (nw * _CHUNK)
    idx3 = src.reshape(nw, nch, _CHUNK)

    g = _sc_gather(tok_embedding, idx3)

    g2 = g.reshape(n // 2, 2 * d)

    pe_flat = pe[0, :seqlen].reshape(seqlen * d // (2 * d), 2 * d)
    block_rows = 6400
    reps = block_rows // pe_flat.shape[0]
    pe_rep = jnp.broadcast_to(pe_flat[None], (reps,) + pe_flat.shape)
    pe_rep = pe_rep.reshape(block_rows, 2 * d)

    wt = W.T
    w2 = jnp.zeros((2 * d, 2 * d), W.dtype)
    w2 = w2.at[:d, :d].set(wt).at[d:, d:].set(wt)
    c2 = jnp.tile(b + 1.0, 2)[None]

    z2 = _tc_finish(g2, pe_rep, w2, c2, block_rows)
    return z2.reshape(bsz, seqlen, d)


# speedup vs baseline: 1.0519x; 1.0519x over previous
import functools
import math

import jax
import jax.numpy as jnp
from jax import lax
from jax.experimental import pallas as pl
from jax.experimental.pallas import tpu as pltpu
from jax.experimental.pallas import tpu_sc as plsc

_CHUNK = 128


def _sc_gather(table, idx3):
    nw, nch, chunk = idx3.shape
    d = table.shape[1]
    lanes_rows = chunk * d // 128
    info = plsc.get_sparse_core_info()
    ncores = info.num_cores

    mesh = plsc.VectorSubcoreMesh(core_axis_name="c", subcore_axis_name="s")

    @functools.partial(
        pl.kernel,
        mesh=mesh,
        compiler_params=pltpu.CompilerParams(use_tc_tiling_on_sc=False),
        out_type=jax.ShapeDtypeStruct((nw, nch * lanes_rows, 128), jnp.float32),
        scratch_types=[
            pltpu.VMEM((nch, chunk), jnp.int32),
            pltpu.VMEM((chunk, d), jnp.float32),
            pltpu.VMEM((chunk, d), jnp.float32),
            pltpu.VMEM((lanes_rows, 128), jnp.float32),
            pltpu.VMEM((lanes_rows, 128), jnp.float32),
            pltpu.SemaphoreType.DMA,
            pltpu.SemaphoreType.DMA,
            pltpu.SemaphoreType.DMA,
            pltpu.SemaphoreType.DMA,
            pltpu.SemaphoreType.DMA,
        ],
    )
    def k(table_hbm, idx_hbm, out_hbm, idx_v, rows0, rows1, pr0, pr1,
          gsem0, gsem1, osem0, osem1, isem):
        wid = lax.axis_index("s") * ncores + lax.axis_index("c")

        pltpu.make_async_copy(idx_hbm.at[wid], idx_v, isem).start()
        pltpu.make_async_copy(idx_hbm.at[wid], idx_v, isem).wait()

        def start_gather(c, buf, sem):
            pltpu.make_async_copy(table_hbm.at[idx_v.at[c]], buf, sem).start()

        def wait_gather(c, buf, sem):
            pltpu.make_async_copy(table_hbm.at[idx_v.at[c]], buf, sem).wait()

        def _out_copy(c, pbuf, sem):
            return pltpu.make_async_copy(
                pbuf,
                out_hbm.at[wid, pl.ds(c * lanes_rows, lanes_rows)],
                sem,
            )

        def pairup(buf, pbuf):
            def body(r2, carry):
                for u in range(d // 16):
                    sl = pl.ds(16 * u, 16)
                    pbuf[r2, sl] = buf[2 * r2, sl]
                    pbuf[r2, pl.ds(d + 16 * u, 16)] = buf[2 * r2 + 1, sl]
                return carry
            lax.fori_loop(0, lanes_rows, body, 0)

        start_gather(0, rows0, gsem0)

        def step(c, buf, gsem, pbuf, osem, nbuf, ngsem):
            @pl.when(c + 1 < nch)
            def _():
                start_gather(c + 1, nbuf, ngsem)
            wait_gather(c, buf, gsem)
            @pl.when(c >= 2)
            def _():
                _out_copy(c - 2, pbuf, osem).wait()
            pairup(buf, pbuf)
            _out_copy(c, pbuf, osem).start()

        def pair(p, carry):
            step(2 * p, rows0, gsem0, pr0, osem0, rows1, gsem1)
            step(2 * p + 1, rows1, gsem1, pr1, osem1, rows0, gsem0)
            return carry

        lax.fori_loop(0, nch // 2, pair, 0)

        _out_copy(nch - 2, pr0, osem0).wait()
        _out_copy(nch - 1, pr1, osem1).wait()

    return k(table, idx3)


def _tc_finish(g2, pe_rep, w2, c2, block_rows, d):
    m = g2.shape[0]
    grid = m // block_rows
    scale = 8.0

    def body(g_ref, pe_ref, w_ref, c_ref, o_ref):
        x = g_ref[...] * scale + pe_ref[...]
        r = jnp.dot(x, w_ref[...], preferred_element_type=jnp.float32)
        z = x * (r + c_ref[...])
        o_ref[0, :, :] = z[:, :d]
        o_ref[1, :, :] = z[:, d:]

    return pl.pallas_call(
        body,
        grid=(grid,),
        in_specs=[
            pl.BlockSpec((block_rows, 2 * d), lambda i: (i, 0)),
            pl.BlockSpec((block_rows, 2 * d), lambda i: (0, 0)),
            pl.BlockSpec((2 * d, 2 * d), lambda i: (0, 0)),
            pl.BlockSpec((1, 2 * d), lambda i: (0, 0)),
        ],
        out_specs=pl.BlockSpec((2, block_rows, d), lambda i: (0, i, 0)),
        out_shape=jax.ShapeDtypeStruct((2, m, d), jnp.float32),
    )(g2, pe_rep, w2, c2)


def kernel(src, tok_embedding, W, b, pe):
    bsz, seqlen = src.shape
    d = tok_embedding.shape[1]
    n = bsz * seqlen

    nw = 32
    nch = n // (nw * _CHUNK)

    src_flat = src.reshape(n)
    idx_pairs = jnp.stack([src_flat[: n // 2], src_flat[n // 2:]], axis=1)
    idx3 = idx_pairs.reshape(nw, nch, _CHUNK)

    g = _sc_gather(tok_embedding, idx3)
    g2 = g.reshape(n // 2, 2 * d)

    pe_pair = jnp.tile(pe[0, :seqlen], (1, 2))
    block_rows = 6400
    reps = block_rows // seqlen
    pe_rep = jnp.broadcast_to(pe_pair[None], (reps, seqlen, 2 * d))
    pe_rep = pe_rep.reshape(block_rows, 2 * d)

    wt = W.T
    w2 = jnp.zeros((2 * d, 2 * d), W.dtype)
    w2 = w2.at[:d, :d].set(wt).at[d:, d:].set(wt)
    c2 = jnp.tile(b + 1.0, 2)[None]

    z3 = _tc_finish(g2, pe_rep, w2, c2, block_rows, d)
    return z3.reshape(bsz, seqlen, d)


# speedup vs baseline: 1.1959x; 1.1369x over previous
import functools
import math

import jax
import jax.numpy as jnp
from jax import lax
from jax.experimental import pallas as pl
from jax.experimental.pallas import tpu as pltpu
from jax.experimental.pallas import tpu_sc as plsc

_CHUNK = 64


def _sc_gather(table, idx4):
    _, nw, nch, chunk = idx4.shape
    d = table.shape[1]
    info = plsc.get_sparse_core_info()
    ncores = info.num_cores

    mesh = plsc.VectorSubcoreMesh(core_axis_name="c", subcore_axis_name="s")

    @functools.partial(
        pl.kernel,
        mesh=mesh,
        compiler_params=pltpu.CompilerParams(use_tc_tiling_on_sc=False),
        out_type=jax.ShapeDtypeStruct((nw, nch * chunk, 2 * d), jnp.float32),
        scratch_types=[
            pltpu.VMEM((2, nch, chunk), jnp.int32),
            pltpu.VMEM((chunk, d), jnp.float32),
            pltpu.VMEM((chunk, d), jnp.float32),
            pltpu.VMEM((chunk, d), jnp.float32),
            pltpu.VMEM((chunk, d), jnp.float32),
            pltpu.VMEM((chunk, 2 * d), jnp.float32),
            pltpu.VMEM((chunk, 2 * d), jnp.float32),
            pltpu.SemaphoreType.DMA,
            pltpu.SemaphoreType.DMA,
            pltpu.SemaphoreType.DMA,
            pltpu.SemaphoreType.DMA,
            pltpu.SemaphoreType.DMA,
            pltpu.SemaphoreType.DMA,
            pltpu.SemaphoreType.DMA,
        ],
    )
    def k(table_hbm, idx_hbm, out_hbm, idx_v, a0, a1, b0, b1, p0, p1,
          gsa0, gsa1, gsb0, gsb1, os0, os1, isem):
        wid = lax.axis_index("s") * ncores + lax.axis_index("c")

        pltpu.make_async_copy(idx_hbm.at[:, wid], idx_v, isem).start()
        pltpu.make_async_copy(idx_hbm.at[:, wid], idx_v, isem).wait()

        def gather(h, c, buf, sem):
            return pltpu.make_async_copy(
                table_hbm.at[idx_v.at[h, c]], buf, sem)

        def out_copy(c, pbuf, sem):
            return pltpu.make_async_copy(
                pbuf, out_hbm.at[wid, pl.ds(c * chunk, chunk)], sem)

        def pairup(abuf, bbuf, pbuf):
            def body(r, carry):
                for u in range(d // 16):
                    sl = pl.ds(16 * u, 16)
                    pbuf[r, sl] = abuf[r, sl]
                    pbuf[r, pl.ds(d + 16 * u, 16)] = bbuf[r, sl]
                return carry
            lax.fori_loop(0, chunk, body, 0)

        gather(0, 0, a0, gsa0).start()
        gather(1, 0, b0, gsb0).start()

        def step(c, abuf, gsa, bbuf, gsb, pbuf, osem, nabuf, ngsa, nbbuf, ngsb):
            @pl.when(c + 1 < nch)
            def _():
                gather(0, c + 1, nabuf, ngsa).start()
                gather(1, c + 1, nbbuf, ngsb).start()
            gather(0, c, abuf, gsa).wait()
            gather(1, c, bbuf, gsb).wait()
            @pl.when(c >= 2)
            def _():
                out_copy(c - 2, pbuf, osem).wait()
            pairup(abuf, bbuf, pbuf)
            out_copy(c, pbuf, osem).start()

        def pair(p, carry):
            step(2 * p, a0, gsa0, b0, gsb0, p0, os0, a1, gsa1, b1, gsb1)
            step(2 * p + 1, a1, gsa1, b1, gsb1, p1, os1, a0, gsa0, b0, gsb0)
            return carry

        lax.fori_loop(0, nch // 2, pair, 0)

        out_copy(nch - 2, p0, os0).wait()
        out_copy(nch - 1, p1, os1).wait()

    return k(table, idx4)


def _tc_finish(g2, pe_runs, w2, c2, lp, d, bsz):
    m = g2.shape[0]
    lhalf = m // bsz
    grid = lhalf // lp
    block_rows = lp * bsz
    scale = 8.0

    def body(g_ref, pe_ref, w_ref, c_ref, o_ref):
        pe_blk = pe_ref[...][0]
        x3 = g_ref[...].reshape(lp, bsz, 2 * d) * scale + pe_blk[:, None, :]
        x = x3.reshape(block_rows, 2 * d)
        r = jnp.dot(x, w_ref[...], preferred_element_type=jnp.float32)
        z = x * (r + c_ref[...])
        z3 = z.reshape(lp, bsz, 2 * d)
        for j in range(lp):
            zj = z3[j]
            o_ref[0, j] = zj[:, :d].T
            o_ref[1, j] = zj[:, d:].T

    return pl.pallas_call(
        body,
        grid=(grid,),
        in_specs=[
            pl.BlockSpec((block_rows, 2 * d), lambda i: (i, 0)),
            pl.BlockSpec((1, lp, 2 * d), lambda i: (i, 0, 0)),
            pl.BlockSpec((2 * d, 2 * d), lambda i: (0, 0)),
            pl.BlockSpec((1, 2 * d), lambda i: (0, 0)),
        ],
        out_specs=pl.BlockSpec((2, lp, d, bsz), lambda i: (0, i, 0, 0)),
        out_shape=jax.ShapeDtypeStruct((2, lhalf, d, bsz), jnp.float32),
    )(g2, pe_runs, w2, c2)


def kernel(src, tok_embedding, W, b, pe):
    bsz, seqlen = src.shape
    d = tok_embedding.shape[1]
    n = bsz * seqlen

    nw = 32
    nch = n // (nw * 2 * _CHUNK)

    idx_l = jnp.transpose(src).reshape(n)
    idx4 = idx_l.reshape(2, nw, nch, _CHUNK)

    g = _sc_gather(tok_embedding, idx4)
    g2 = g.reshape(n // 2, 2 * d)

    lp = 2
    pe_runs = jnp.concatenate(
        [pe[0, : seqlen // 2], pe[0, seqlen // 2:]], axis=1)
    pe_runs = pe_runs.reshape(seqlen // 2 // lp, lp, 2 * d)

    wt = W.T
    w2 = jnp.zeros((2 * d, 2 * d), W.dtype)
    w2 = w2.at[:d, :d].set(wt).at[d:, d:].set(wt)
    c2 = jnp.tile(b + 1.0, 2)[None]

    z4 = _tc_finish(g2, pe_runs, w2, c2, lp, d, bsz)
    return jnp.transpose(z4.reshape(seqlen, d, bsz), (2, 0, 1))


# speedup vs baseline: 1.1991x; 1.0027x over previous
import functools
import math

import jax
import jax.numpy as jnp
from jax import lax
from jax.experimental import pallas as pl
from jax.experimental.pallas import tpu as pltpu
from jax.experimental.pallas import tpu_sc as plsc

_CHUNK = 64


def _sc_gather(table, idx2):
    nrows = idx2.shape[0]
    d = table.shape[1]
    nw = 32
    nch = nrows // (2 * nw)
    chunk = idx2.shape[1]
    info = plsc.get_sparse_core_info()
    ncores = info.num_cores

    mesh = plsc.VectorSubcoreMesh(core_axis_name="c", subcore_axis_name="s")

    @functools.partial(
        pl.kernel,
        mesh=mesh,
        compiler_params=pltpu.CompilerParams(use_tc_tiling_on_sc=False),
        out_type=jax.ShapeDtypeStruct((nw, nch * chunk, 2 * d), jnp.float32),
        scratch_types=[
            pltpu.VMEM((nch, chunk), jnp.int32),
            pltpu.VMEM((nch, chunk), jnp.int32),
            pltpu.VMEM((chunk, d), jnp.float32),
            pltpu.VMEM((chunk, d), jnp.float32),
            pltpu.VMEM((chunk, d), jnp.float32),
            pltpu.VMEM((chunk, d), jnp.float32),
            pltpu.VMEM((chunk, 2 * d), jnp.float32),
            pltpu.VMEM((chunk, 2 * d), jnp.float32),
            pltpu.SemaphoreType.DMA,
            pltpu.SemaphoreType.DMA,
            pltpu.SemaphoreType.DMA,
            pltpu.SemaphoreType.DMA,
            pltpu.SemaphoreType.DMA,
            pltpu.SemaphoreType.DMA,
            pltpu.SemaphoreType.DMA,
        ],
    )
    def k(table_hbm, idx_hbm, out_hbm, idx_va, idx_vb, a0, a1, b0, b1, p0, p1,
          gsa0, gsa1, gsb0, gsb1, os0, os1, isem):
        wid = lax.axis_index("s") * ncores + lax.axis_index("c")

        pltpu.make_async_copy(
            idx_hbm.at[pl.ds(wid * nch, nch)], idx_va, isem).start()
        pltpu.make_async_copy(
            idx_hbm.at[pl.ds(nrows // 2 + wid * nch, nch)], idx_vb, isem).start()
        pltpu.make_async_copy(
            idx_hbm.at[pl.ds(wid * nch, nch)], idx_va, isem).wait()
        pltpu.make_async_copy(
            idx_hbm.at[pl.ds(wid * nch, nch)], idx_vb, isem).wait()

        def gather(idx_v, c, buf, sem):
            return pltpu.make_async_copy(
                table_hbm.at[idx_v.at[c]], buf, sem)

        def out_copy(c, pbuf, sem):
            return pltpu.make_async_copy(
                pbuf, out_hbm.at[wid, pl.ds(c * chunk, chunk)], sem)

        def pairup(abuf, bbuf, pbuf):
            def body(r, carry):
                for u in range(d // 16):
                    sl = pl.ds(16 * u, 16)
                    pbuf[r, sl] = abuf[r, sl]
                    pbuf[r, pl.ds(d + 16 * u, 16)] = bbuf[r, sl]
                return carry
            lax.fori_loop(0, chunk, body, 0)

        gather(idx_va, 0, a0, gsa0).start()
        gather(idx_vb, 0, b0, gsb0).start()

        def step(c, abuf, gsa, bbuf, gsb, pbuf, osem, nabuf, ngsa, nbbuf, ngsb):
            @pl.when(c + 1 < nch)
            def _():
                gather(idx_va, c + 1, nabuf, ngsa).start()
                gather(idx_vb, c + 1, nbbuf, ngsb).start()
            gather(idx_va, c, abuf, gsa).wait()
            gather(idx_vb, c, bbuf, gsb).wait()
            @pl.when(c >= 2)
            def _():
                out_copy(c - 2, pbuf, osem).wait()
            pairup(abuf, bbuf, pbuf)
            out_copy(c, pbuf, osem).start()

        def pair(p, carry):
            step(2 * p, a0, gsa0, b0, gsb0, p0, os0, a1, gsa1, b1, gsb1)
            step(2 * p + 1, a1, gsa1, b1, gsb1, p1, os1, a0, gsa0, b0, gsb0)
            return carry

        lax.fori_loop(0, nch // 2, pair, 0)

        out_copy(nch - 2, p0, os0).wait()
        out_copy(nch - 1, p1, os1).wait()

    return k(table, idx2)


def _tc_finish(g2, pe_runs, w2, c2, lp, d, bsz):
    m = g2.shape[0]
    lhalf = m // bsz
    grid = lhalf // lp
    block_rows = lp * bsz
    scale = 8.0

    def body(g_ref, pe_ref, w_ref, c_ref, o_ref):
        pe_blk = pe_ref[...][0]
        x3 = g_ref[...].reshape(lp, bsz, 2 * d) * scale + pe_blk[:, None, :]
        x = x3.reshape(block_rows, 2 * d)
        r = jnp.dot(x, w_ref[...], preferred_element_type=jnp.float32)
        z = x * (r + c_ref[...])
        z3 = z.reshape(lp, bsz, 2 * d)
        for j in range(lp):
            zj = z3[j]
            o_ref[0, j] = zj[:, :d].T
            o_ref[1, j] = zj[:, d:].T

    return pl.pallas_call(
        body,
        grid=(grid,),
        in_specs=[
            pl.BlockSpec((block_rows, 2 * d), lambda i: (i, 0)),
            pl.BlockSpec((1, lp, 2 * d), lambda i: (i, 0, 0)),
            pl.BlockSpec((2 * d, 2 * d), lambda i: (0, 0)),
            pl.BlockSpec((1, 2 * d), lambda i: (0, 0)),
        ],
        out_specs=pl.BlockSpec((2, lp, d, bsz), lambda i: (0, i, 0, 0)),
        out_shape=jax.ShapeDtypeStruct((2, lhalf, d, bsz), jnp.float32),
    )(g2, pe_runs, w2, c2)


def kernel(src, tok_embedding, W, b, pe):
    bsz, seqlen = src.shape
    d = tok_embedding.shape[1]
    n = bsz * seqlen

    idx2 = jnp.transpose(src).reshape(n // 128, 128)

    g = _sc_gather(tok_embedding, idx2)
    g2 = g.reshape(n // 2, 2 * d)

    lp = 2
    pe_runs = jnp.concatenate(
        [pe[0, : seqlen // 2], pe[0, seqlen // 2:]], axis=1)
    pe_runs = pe_runs.reshape(seqlen // 2 // lp, lp, 2 * d)

    wt = W.T
    w2 = jnp.zeros((2 * d, 2 * d), W.dtype)
    w2 = w2.at[:d, :d].set(wt).at[d:, d:].set(wt)
    c2 = jnp.tile(b + 1.0, 2)[None]

    z4 = _tc_finish(g2, pe_runs, w2, c2, lp, d, bsz)
    return jnp.transpose(z4.reshape(seqlen, d, bsz), (2, 0, 1))


# speedup vs baseline: 1.4787x; 1.2332x over previous
import functools
import math

import jax
import jax.numpy as jnp
from jax import lax
from jax.experimental import pallas as pl
from jax.experimental.pallas import tpu as pltpu
from jax.experimental.pallas import tpu_sc as plsc

_CHUNK = 64


def _sc_gather(table, idx2):
    nrows = idx2.shape[0]
    d = table.shape[1]
    nw = 32
    nch = nrows // (2 * nw)
    chunk = idx2.shape[1]
    info = plsc.get_sparse_core_info()
    ncores = info.num_cores

    mesh = plsc.VectorSubcoreMesh(core_axis_name="c", subcore_axis_name="s")

    @functools.partial(
        pl.kernel,
        mesh=mesh,
        compiler_params=pltpu.CompilerParams(use_tc_tiling_on_sc=False),
        out_type=jax.ShapeDtypeStruct((nw, nch * chunk, 2 * d), jnp.float32),
        scratch_types=[
            pltpu.VMEM((nch, chunk), jnp.int32),
            pltpu.VMEM((nch, chunk), jnp.int32),
            pltpu.VMEM((chunk, d), jnp.float32),
            pltpu.VMEM((chunk, d), jnp.float32),
            pltpu.VMEM((chunk, d), jnp.float32),
            pltpu.VMEM((chunk, d), jnp.float32),
            pltpu.VMEM((chunk, 2 * d), jnp.float32),
            pltpu.VMEM((chunk, 2 * d), jnp.float32),
            pltpu.SemaphoreType.DMA,
            pltpu.SemaphoreType.DMA,
            pltpu.SemaphoreType.DMA,
            pltpu.SemaphoreType.DMA,
            pltpu.SemaphoreType.DMA,
            pltpu.SemaphoreType.DMA,
            pltpu.SemaphoreType.DMA,
        ],
    )
    def k(table_hbm, idx_hbm, out_hbm, idx_va, idx_vb, a0, a1, b0, b1, p0, p1,
          gsa0, gsa1, gsb0, gsb1, os0, os1, isem):
        wid = lax.axis_index("s") * ncores + lax.axis_index("c")

        pltpu.make_async_copy(
            idx_hbm.at[pl.ds(wid * nch, nch)], idx_va, isem).start()
        pltpu.make_async_copy(
            idx_hbm.at[pl.ds(nrows // 2 + wid * nch, nch)], idx_vb, isem).start()
        pltpu.make_async_copy(
            idx_hbm.at[pl.ds(wid * nch, nch)], idx_va, isem).wait()
        pltpu.make_async_copy(
            idx_hbm.at[pl.ds(wid * nch, nch)], idx_vb, isem).wait()

        def gather(idx_v, c, buf, sem):
            return pltpu.make_async_copy(
                table_hbm.at[idx_v.at[c]], buf, sem)

        def out_copy(c, pbuf, sem):
            return pltpu.make_async_copy(
                pbuf, out_hbm.at[wid, pl.ds(c * chunk, chunk)], sem)

        def pairup(abuf, bbuf, pbuf):
            nu = d // 16

            def body(r, carry):
                va = [abuf[r, pl.ds(16 * u, 16)] for u in range(nu)]
                vb = [bbuf[r, pl.ds(16 * u, 16)] for u in range(nu)]
                for u in range(nu):
                    pbuf[r, pl.ds(16 * u, 16)] = va[u]
                for u in range(nu):
                    pbuf[r, pl.ds(d + 16 * u, 16)] = vb[u]
                return carry
            lax.fori_loop(0, chunk, body, 0)

        gather(idx_va, 0, a0, gsa0).start()
        gather(idx_vb, 0, b0, gsb0).start()

        def step(c, abuf, gsa, bbuf, gsb, pbuf, osem, nabuf, ngsa, nbbuf, ngsb):
            @pl.when(c + 1 < nch)
            def _():
                gather(idx_va, c + 1, nabuf, ngsa).start()
                gather(idx_vb, c + 1, nbbuf, ngsb).start()
            gather(idx_va, c, abuf, gsa).wait()
            gather(idx_vb, c, bbuf, gsb).wait()
            @pl.when(c >= 2)
            def _():
                out_copy(c - 2, pbuf, osem).wait()
            pairup(abuf, bbuf, pbuf)
            out_copy(c, pbuf, osem).start()

        def pair(p, carry):
            step(2 * p, a0, gsa0, b0, gsb0, p0, os0, a1, gsa1, b1, gsb1)
            step(2 * p + 1, a1, gsa1, b1, gsb1, p1, os1, a0, gsa0, b0, gsb0)
            return carry

        lax.fori_loop(0, nch // 2, pair, 0)

        out_copy(nch - 2, p0, os0).wait()
        out_copy(nch - 1, p1, os1).wait()

    return k(table, idx2)


def _tc_finish(g2, pe_runs, w2, c2, lp, d, bsz):
    m = g2.shape[0]
    lhalf = m // bsz
    grid = lhalf // lp
    block_rows = lp * bsz
    scale = 8.0

    def body(g_ref, pe_ref, w_ref, c_ref, o_ref):
        pe_blk = pe_ref[...][0]
        x3 = g_ref[...].reshape(lp, bsz, 2 * d) * scale + pe_blk[:, None, :]
        x = x3.reshape(block_rows, 2 * d)
        r = jnp.dot(x, w_ref[...], preferred_element_type=jnp.float32)
        z = x * (r + c_ref[...])
        z3 = z.reshape(lp, bsz, 2 * d)
        for j in range(lp):
            zj = z3[j]
            o_ref[0, j] = zj[:, :d].T
            o_ref[1, j] = zj[:, d:].T

    return pl.pallas_call(
        body,
        grid=(grid,),
        in_specs=[
            pl.BlockSpec((block_rows, 2 * d), lambda i: (i, 0)),
            pl.BlockSpec((1, lp, 2 * d), lambda i: (i, 0, 0)),
            pl.BlockSpec((2 * d, 2 * d), lambda i: (0, 0)),
            pl.BlockSpec((1, 2 * d), lambda i: (0, 0)),
        ],
        out_specs=pl.BlockSpec((2, lp, d, bsz), lambda i: (0, i, 0, 0)),
        out_shape=jax.ShapeDtypeStruct((2, lhalf, d, bsz), jnp.float32),
    )(g2, pe_runs, w2, c2)


def kernel(src, tok_embedding, W, b, pe):
    bsz, seqlen = src.shape
    d = tok_embedding.shape[1]
    n = bsz * seqlen

    idx2 = jnp.transpose(src).reshape(n // 128, 128)

    g = _sc_gather(tok_embedding, idx2)
    g2 = g.reshape(n // 2, 2 * d)

    lp = 2
    pe_runs = jnp.concatenate(
        [pe[0, : seqlen // 2], pe[0, seqlen // 2:]], axis=1)
    pe_runs = pe_runs.reshape(seqlen // 2 // lp, lp, 2 * d)

    wt = W.T
    w2 = jnp.zeros((2 * d, 2 * d), W.dtype)
    w2 = w2.at[:d, :d].set(wt).at[d:, d:].set(wt)
    c2 = jnp.tile(b + 1.0, 2)[None]

    z4 = _tc_finish(g2, pe_runs, w2, c2, lp, d, bsz)
    return jnp.transpose(z4.reshape(seqlen, d, bsz), (2, 0, 1))


# speedup vs baseline: 1.4933x; 1.0099x over previous
import functools
import math

import jax
import jax.numpy as jnp
from jax import lax
from jax.experimental import pallas as pl
from jax.experimental.pallas import tpu as pltpu
from jax.experimental.pallas import tpu_sc as plsc

_CHUNK = 64


def _sc_gather(table, idx2):
    nrows = idx2.shape[0]
    d = table.shape[1]
    nw = 32
    nch = nrows // (2 * nw)
    chunk = idx2.shape[1]
    info = plsc.get_sparse_core_info()
    ncores = info.num_cores

    mesh = plsc.VectorSubcoreMesh(core_axis_name="c", subcore_axis_name="s")

    @functools.partial(
        pl.kernel,
        mesh=mesh,
        compiler_params=pltpu.CompilerParams(use_tc_tiling_on_sc=False),
        out_type=jax.ShapeDtypeStruct((nw, nch * chunk, 2 * d), jnp.float32),
        scratch_types=[
            pltpu.VMEM((nch, chunk), jnp.int32),
            pltpu.VMEM((nch, chunk), jnp.int32),
            pltpu.VMEM((chunk, d), jnp.float32),
            pltpu.VMEM((chunk, d), jnp.float32),
            pltpu.VMEM((chunk, d), jnp.float32),
            pltpu.VMEM((chunk, d), jnp.float32),
            pltpu.VMEM((chunk, 2 * d), jnp.float32),
            pltpu.VMEM((chunk, 2 * d), jnp.float32),
            pltpu.SemaphoreType.DMA,
            pltpu.SemaphoreType.DMA,
            pltpu.SemaphoreType.DMA,
            pltpu.SemaphoreType.DMA,
            pltpu.SemaphoreType.DMA,
            pltpu.SemaphoreType.DMA,
            pltpu.SemaphoreType.DMA,
        ],
    )
    def k(table_hbm, idx_hbm, out_hbm, idx_va, idx_vb, a0, a1, b0, b1, p0, p1,
          gsa0, gsa1, gsb0, gsb1, os0, os1, isem):
        wid = lax.axis_index("s") * ncores + lax.axis_index("c")

        pltpu.make_async_copy(
            idx_hbm.at[pl.ds(wid * nch, nch)], idx_va, isem).start()
        pltpu.make_async_copy(
            idx_hbm.at[pl.ds(nrows // 2 + wid * nch, nch)], idx_vb, isem).start()
        pltpu.make_async_copy(
            idx_hbm.at[pl.ds(wid * nch, nch)], idx_va, isem).wait()
        pltpu.make_async_copy(
            idx_hbm.at[pl.ds(wid * nch, nch)], idx_vb, isem).wait()

        def gather(idx_v, c, buf, sem):
            return pltpu.make_async_copy(
                table_hbm.at[idx_v.at[c]], buf, sem)

        def out_copy(c, pbuf, sem):
            return pltpu.make_async_copy(
                pbuf, out_hbm.at[wid, pl.ds(c * chunk, chunk)], sem)

        def pairup(abuf, bbuf, pbuf):
            nu = d // 16

            def body(r, carry):
                va = [abuf[r, pl.ds(16 * u, 16)] for u in range(nu)]
                vb = [bbuf[r, pl.ds(16 * u, 16)] for u in range(nu)]
                for u in range(nu):
                    pbuf[r, pl.ds(16 * u, 16)] = va[u]
                for u in range(nu):
                    pbuf[r, pl.ds(d + 16 * u, 16)] = vb[u]
                return carry
            lax.fori_loop(0, chunk, body, 0)

        gather(idx_va, 0, a0, gsa0).start()
        gather(idx_vb, 0, b0, gsb0).start()

        def step(c, abuf, gsa, bbuf, gsb, pbuf, osem, nabuf, ngsa, nbbuf, ngsb):
            @pl.when(c + 1 < nch)
            def _():
                gather(idx_va, c + 1, nabuf, ngsa).start()
                gather(idx_vb, c + 1, nbbuf, ngsb).start()
            gather(idx_va, c, abuf, gsa).wait()
            gather(idx_vb, c, bbuf, gsb).wait()
            @pl.when(c >= 2)
            def _():
                out_copy(c - 2, pbuf, osem).wait()
            pairup(abuf, bbuf, pbuf)
            out_copy(c, pbuf, osem).start()

        def pair(p, carry):
            step(2 * p, a0, gsa0, b0, gsb0, p0, os0, a1, gsa1, b1, gsb1)
            step(2 * p + 1, a1, gsa1, b1, gsb1, p1, os1, a0, gsa0, b0, gsb0)
            return carry

        lax.fori_loop(0, nch // 2, pair, 0)

        out_copy(nch - 2, p0, os0).wait()
        out_copy(nch - 1, p1, os1).wait()

    return k(table, idx2)


def _tc_finish(g2, pe_runs, w2, c2, lp, d, bsz):
    m = g2.shape[0]
    lhalf = m // bsz
    grid = lhalf // lp
    block_rows = lp * bsz
    scale = 8.0

    def body(g_ref, pe_ref, w_ref, c_ref, o_ref):
        pe_blk = pe_ref[...][0]
        x3 = g_ref[...].reshape(lp, bsz, 2 * d) * scale + pe_blk[:, None, :]
        x = x3.reshape(block_rows, 2 * d)
        r = jnp.dot(x, w_ref[...], preferred_element_type=jnp.float32)
        z = x * (r + c_ref[...])
        z3 = z.reshape(lp, bsz, 2 * d)
        for j in range(lp):
            zj = z3[j]
            o_ref[0, j] = zj[:, :d].T
            o_ref[1, j] = zj[:, d:].T

    return pl.pallas_call(
        body,
        grid=(grid,),
        in_specs=[
            pl.BlockSpec((block_rows, 2 * d), lambda i: (i, 0)),
            pl.BlockSpec((1, lp, 2 * d), lambda i: (i, 0, 0)),
            pl.BlockSpec((2 * d, 2 * d), lambda i: (0, 0)),
            pl.BlockSpec((1, 2 * d), lambda i: (0, 0)),
        ],
        out_specs=pl.BlockSpec((2, lp, d, bsz), lambda i: (0, i, 0, 0)),
        out_shape=jax.ShapeDtypeStruct((2, lhalf, d, bsz), jnp.float32),
    )(g2, pe_runs, w2, c2)


def kernel(src, tok_embedding, W, b, pe):
    bsz, seqlen = src.shape
    d = tok_embedding.shape[1]
    n = bsz * seqlen

    idx2 = jnp.transpose(src).reshape(n // 128, 128)

    g = _sc_gather(tok_embedding, idx2)
    g2 = g.reshape(n // 2, 2 * d)

    lp = 4
    pe_runs = jnp.concatenate(
        [pe[0, : seqlen // 2], pe[0, seqlen // 2:]], axis=1)
    pe_runs = pe_runs.reshape(seqlen // 2 // lp, lp, 2 * d)

    wt = W.T
    w2 = jnp.zeros((2 * d, 2 * d), W.dtype)
    w2 = w2.at[:d, :d].set(wt).at[d:, d:].set(wt)
    c2 = jnp.tile(b + 1.0, 2)[None]

    z4 = _tc_finish(g2, pe_runs, w2, c2, lp, d, bsz)
    return jnp.transpose(z4.reshape(seqlen, d, bsz), (2, 0, 1))
